# baseline (speedup 1.0000x reference)
import functools

import jax
import jax.numpy as jnp
from jax.experimental import pallas as pl
from jax.experimental.pallas import tpu as pltpu

TOKENS = 32768
DIM = 768
N_EXPERTS = 8
CHUNK = 4096
NCHUNKS = TOKENS // CHUNK
NBUF = 4


def _gate_kernel(x_hbm, w_ref, w_out_ref, idx_out_ref, xbuf, copy_sem):
    def start(c):
        pltpu.make_async_copy(
            x_hbm.at[pl.ds(c * CHUNK, CHUNK), :],
            xbuf.at[c % NBUF],
            copy_sem.at[c % NBUF],
        ).start()

    for c in range(min(NBUF, NCHUNKS)):
        start(c)

    w = w_ref[...]
    for c in range(NCHUNKS):
        pltpu.make_async_copy(
            x_hbm.at[pl.ds(c * CHUNK, CHUNK), :],
            xbuf.at[c % NBUF],
            copy_sem.at[c % NBUF],
        ).wait()
        s = jax.lax.dot_general(
            xbuf[c % NBUF], w,
            dimension_numbers=(((1,), (1,)), ((), ())),
            preferred_element_type=jnp.float32)
        if c + NBUF < NCHUNKS:
            start(c + NBUF)
        st = s.T
        m = jnp.max(st, axis=0, keepdims=True)
        denom = jnp.sum(jnp.exp(st - m), axis=0, keepdims=True)
        w_out_ref[pl.ds(c * CHUNK, CHUNK)] = (1.0 / denom).reshape(CHUNK)
        idx_out_ref[pl.ds(c * CHUNK, CHUNK)] = (
            jnp.argmax(st, axis=0).reshape(CHUNK).astype(jnp.int32))


@jax.jit
def kernel(x, weight):
    weights, indices = pl.pallas_call(
        _gate_kernel,
        in_specs=[
            pl.BlockSpec(memory_space=pltpu.HBM),
            pl.BlockSpec(memory_space=pltpu.VMEM),
        ],
        out_specs=[
            pl.BlockSpec(memory_space=pltpu.VMEM),
            pl.BlockSpec(memory_space=pltpu.VMEM),
        ],
        out_shape=[
            jax.ShapeDtypeStruct((TOKENS,), jnp.float32),
            jax.ShapeDtypeStruct((TOKENS,), jnp.int32),
        ],
        scratch_shapes=[
            pltpu.VMEM((NBUF, CHUNK, DIM), jnp.float32),
            pltpu.SemaphoreType.DMA((NBUF,)),
        ],
    )(x, weight)
    return weights.reshape(TOKENS, 1), indices.reshape(TOKENS, 1)
